# baseline (speedup 1.0000x reference)
import functools

import jax
import jax.numpy as jnp
from jax import lax
from jax.experimental import pallas as pl
from jax.experimental.pallas import tpu as pltpu
from jax.experimental.pallas import tpu_sc as plsc

N = 19
E = 342
EP = 384
D_IN = 1025
HID = 256
OUT = 512
AP = 512


def _build_a_body(dst_h, src_h, pc_h, out_h,
                  dst_v, src_v, idx_v, pc_v, buf_v, acc_sh, sem):
    del sem
    cid = lax.axis_index("c")
    sid = lax.axis_index("s")

    @pl.when((cid == 0) & (sid == 0))
    def _():
        pltpu.sync_copy(dst_h, dst_v)
        pltpu.sync_copy(src_h, src_v)
        pltpu.sync_copy(pc_h, pc_v)
        for i in range(AP // 16):
            buf_v[pl.ds(i * 16, 16)] = jnp.zeros((16,), jnp.float32)
        for e in range(EP // 16):
            d = dst_v[pl.ds(e * 16, 16)]
            s = src_v[pl.ds(e * 16, 16)]
            idx_v[e // 8, pl.ds((e % 8) * 16, 16)] = d * N + s
        pltpu.sync_copy(buf_v, acc_sh)
        for j in range(EP // 128):
            pltpu.sync_copy(pc_v.at[j], acc_sh.at[idx_v.at[j]], add=True)
        pltpu.sync_copy(acc_sh, out_h)


_build_a = functools.partial(
    pl.kernel,
    out_type=jax.ShapeDtypeStruct((AP,), jnp.float32),
    mesh=plsc.VectorSubcoreMesh(core_axis_name="c", subcore_axis_name="s"),
    scratch_types=[
        pltpu.VMEM((EP,), jnp.int32),
        pltpu.VMEM((EP,), jnp.int32),
        pltpu.VMEM((EP // 128, 128), jnp.int32),
        pltpu.VMEM((EP // 128, 128), jnp.float32),
        pltpu.VMEM((AP,), jnp.float32),
        pltpu.VMEM_SHARED((AP,), jnp.float32),
        pltpu.SemaphoreType.DMA,
    ],
)(_build_a_body)


def _dense_kernel(a_ref, x_ref,
                  w1a_ref, b1a_ref, w1b_ref, b1b_ref,
                  w2a_ref, b2a_ref, w2b_ref, b2b_ref,
                  wp_ref, bp_ref, out_ref):
    f32 = jnp.float32
    eye = (jax.lax.broadcasted_iota(jnp.int32, (N, N), 0)
           == jax.lax.broadcasted_iota(jnp.int32, (N, N), 1)).astype(f32)
    apl = a_ref[...] + eye

    z1 = jnp.dot(apl, x_ref[...], preferred_element_type=f32)
    t1 = jax.nn.relu(jnp.dot(z1, w1a_ref[...], preferred_element_type=f32)
                     + b1a_ref[...])
    h1 = jax.nn.relu(jnp.dot(t1, w1b_ref[...], preferred_element_type=f32)
                     + b1b_ref[...])

    z2 = jnp.dot(apl, h1, preferred_element_type=f32)
    t2 = jax.nn.relu(jnp.dot(z2, w2a_ref[...], preferred_element_type=f32)
                     + b2a_ref[...])
    h2 = jnp.dot(t2, w2b_ref[...], preferred_element_type=f32) + b2b_ref[...]

    s = jnp.sum(h2 * wp_ref[...], axis=1, keepdims=True)
    total = jnp.sum(s, axis=0, keepdims=True) + bp_ref[...]
    out_ref[...] = jax.nn.sigmoid(total)


@jax.jit
def _run(x, edge_index, pred_connectivity, W1a, b1a, W1b, b1b,
         W2a, b2a, W2b, b2b, Wp, bp):
    src = edge_index[0]
    dst = edge_index[1]
    pad = EP - E
    dst_p = jnp.pad(dst, (0, pad))
    src_p = jnp.pad(src, (0, pad))
    pc_p = jnp.pad(pred_connectivity, (0, pad)).reshape(EP // 128, 128)

    a_flat = _build_a(dst_p, src_p, pc_p)
    a2d = a_flat[:N * N].reshape(N, N)

    out = pl.pallas_call(
        _dense_kernel,
        out_shape=jax.ShapeDtypeStruct((1, 1), jnp.float32),
    )(a2d, x,
      W1a, b1a.reshape(1, HID), W1b, b1b.reshape(1, HID),
      W2a, b2a.reshape(1, OUT), W2b, b2b.reshape(1, OUT),
      Wp.reshape(N, OUT), bp.reshape(1, 1))
    return out.reshape(1)


def kernel(x, edge_index, pred_connectivity, W1a, b1a, W1b, b1b,
           W2a, b2a, W2b, b2b, Wp, bp):
    return _run(x, edge_index, pred_connectivity, W1a, b1a, W1b, b1b,
                W2a, b2a, W2b, b2b, Wp, bp)


# speedup vs baseline: 1.0184x; 1.0184x over previous
import functools

import jax
import jax.numpy as jnp
from jax import lax
from jax.experimental import pallas as pl
from jax.experimental.pallas import tpu as pltpu
from jax.experimental.pallas import tpu_sc as plsc

N = 19
E = 342
EP = 384
D_IN = 1025
HID = 256
OUT = 512
AP = 512


def _build_a_body(dst_h, src_h, pc_h, out_h,
                  dst_v, src_v, idx_v, pc_v, buf_v, acc_sh, sem):
    del sem
    cid = lax.axis_index("c")
    sid = lax.axis_index("s")

    @pl.when((cid == 0) & (sid == 0))
    def _():
        pltpu.sync_copy(dst_h, dst_v)
        pltpu.sync_copy(src_h, src_v)
        pltpu.sync_copy(pc_h, pc_v)
        for i in range(AP // 16):
            buf_v[pl.ds(i * 16, 16)] = jnp.zeros((16,), jnp.float32)
        for e in range(EP // 16):
            d = dst_v[pl.ds(e * 16, 16)]
            s = src_v[pl.ds(e * 16, 16)]
            idx_v[e // 8, pl.ds((e % 8) * 16, 16)] = d * N + s
        pltpu.sync_copy(buf_v, acc_sh)
        for j in range(EP // 128):
            pltpu.sync_copy(pc_v.at[j], acc_sh.at[idx_v.at[j]], add=True)
        pltpu.sync_copy(acc_sh, out_h)


_build_a = functools.partial(
    pl.kernel,
    out_type=jax.ShapeDtypeStruct((AP,), jnp.float32),
    mesh=plsc.VectorSubcoreMesh(core_axis_name="c", subcore_axis_name="s"),
    scratch_types=[
        pltpu.VMEM((EP,), jnp.int32),
        pltpu.VMEM((EP,), jnp.int32),
        pltpu.VMEM((EP // 128, 128), jnp.int32),
        pltpu.VMEM((EP // 128, 128), jnp.float32),
        pltpu.VMEM((AP,), jnp.float32),
        pltpu.VMEM_SHARED((AP,), jnp.float32),
        pltpu.SemaphoreType.DMA,
    ],
)(_build_a_body)


def _proj_kernel(x_ref, w1a_ref, out_ref):
    out_ref[...] = jnp.dot(x_ref[...], w1a_ref[...],
                           preferred_element_type=jnp.float32)


def _dense_kernel(a_ref, p_ref,
                  b1a_ref, w1b_ref, b1b_ref,
                  w2a_ref, b2a_ref, w2b_ref, b2b_ref,
                  wp_ref, bp_ref, out_ref):
    f32 = jnp.float32
    eye = (jax.lax.broadcasted_iota(jnp.int32, (N, N), 0)
           == jax.lax.broadcasted_iota(jnp.int32, (N, N), 1)).astype(f32)
    apl = a_ref[...] + eye

    t1 = jax.nn.relu(jnp.dot(apl, p_ref[...], preferred_element_type=f32)
                     + b1a_ref[...])
    h1 = jax.nn.relu(jnp.dot(t1, w1b_ref[...], preferred_element_type=f32)
                     + b1b_ref[...])

    z2 = jnp.dot(apl, h1, preferred_element_type=f32)
    t2 = jax.nn.relu(jnp.dot(z2, w2a_ref[...], preferred_element_type=f32)
                     + b2a_ref[...])
    h2 = jnp.dot(t2, w2b_ref[...], preferred_element_type=f32) + b2b_ref[...]

    s = jnp.sum(h2 * wp_ref[...], axis=1, keepdims=True)
    total = jnp.sum(s, axis=0, keepdims=True) + bp_ref[...]
    out_ref[...] = jax.nn.sigmoid(total)


@jax.jit
def _run(x, edge_index, pred_connectivity, W1a, b1a, W1b, b1b,
         W2a, b2a, W2b, b2b, Wp, bp):
    src = edge_index[0]
    dst = edge_index[1]
    pad = EP - E
    dst_p = jnp.pad(dst, (0, pad))
    src_p = jnp.pad(src, (0, pad))
    pc_p = jnp.pad(pred_connectivity, (0, pad)).reshape(EP // 128, 128)

    a_flat = _build_a(dst_p, src_p, pc_p)
    p = pl.pallas_call(
        _proj_kernel,
        out_shape=jax.ShapeDtypeStruct((N, HID), jnp.float32),
    )(x, W1a)
    a2d = a_flat[:N * N].reshape(N, N)

    out = pl.pallas_call(
        _dense_kernel,
        out_shape=jax.ShapeDtypeStruct((1, 1), jnp.float32),
    )(a2d, p,
      b1a.reshape(1, HID), W1b, b1b.reshape(1, HID),
      W2a, b2a.reshape(1, OUT), W2b, b2b.reshape(1, OUT),
      Wp.reshape(N, OUT), bp.reshape(1, 1))
    return out.reshape(1)


def kernel(x, edge_index, pred_connectivity, W1a, b1a, W1b, b1b,
           W2a, b2a, W2b, b2b, Wp, bp):
    return _run(x, edge_index, pred_connectivity, W1a, b1a, W1b, b1b,
                W2a, b2a, W2b, b2b, Wp, bp)


# speedup vs baseline: 1.0978x; 1.0779x over previous
import functools

import jax
import jax.numpy as jnp
from jax import lax
from jax.experimental import pallas as pl
from jax.experimental.pallas import tpu as pltpu
from jax.experimental.pallas import tpu_sc as plsc

N = 19
E = 342
EP = 384
D_IN = 1025
HID = 256
OUT = 512
AP = 512


def _build_a_body(ds_h, pc_h, out_h,
                  ds_v, idx_v, pc_v, buf_v, acc_sh, sem):
    del sem
    cid = lax.axis_index("c")
    sid = lax.axis_index("s")

    @pl.when((cid == 0) & (sid == 0))
    def _():
        pltpu.sync_copy(ds_h, ds_v)
        pltpu.sync_copy(pc_h, pc_v)
        for i in range(AP // 16):
            buf_v[pl.ds(i * 16, 16)] = jnp.zeros((16,), jnp.float32)
        for e in range(EP // 16):
            d = ds_v[pl.ds(e * 16, 16)]
            s = ds_v[pl.ds(EP + e * 16, 16)]
            idx_v[e // 8, pl.ds((e % 8) * 16, 16)] = d * N + s
        pltpu.sync_copy(buf_v, acc_sh)
        for j in range(EP // 128):
            pltpu.sync_copy(pc_v.at[j], acc_sh.at[idx_v.at[j]], add=True)
        pltpu.sync_copy(acc_sh, out_h)


_build_a = functools.partial(
    pl.kernel,
    out_type=jax.ShapeDtypeStruct((AP,), jnp.float32),
    mesh=plsc.VectorSubcoreMesh(core_axis_name="c", subcore_axis_name="s",
                                num_cores=1),
    scratch_types=[
        pltpu.VMEM((2 * EP,), jnp.int32),
        pltpu.VMEM((EP // 128, 128), jnp.int32),
        pltpu.VMEM((EP // 128, 128), jnp.float32),
        pltpu.VMEM((AP,), jnp.float32),
        pltpu.VMEM_SHARED((AP,), jnp.float32),
        pltpu.SemaphoreType.DMA,
    ],
)(_build_a_body)


def _proj_kernel(x_ref, w1a_ref, out_ref):
    out_ref[...] = jnp.dot(x_ref[...], w1a_ref[...],
                           preferred_element_type=jnp.float32)


def _dense_kernel(a_ref, p_ref,
                  b1a_ref, w1b_ref, b1b_ref,
                  w2a_ref, b2a_ref, w2b_ref, b2b_ref,
                  wp_ref, bp_ref, out_ref):
    f32 = jnp.float32
    eye = (jax.lax.broadcasted_iota(jnp.int32, (N, N), 0)
           == jax.lax.broadcasted_iota(jnp.int32, (N, N), 1)).astype(f32)
    apl = a_ref[...] + eye

    t1 = jax.nn.relu(jnp.dot(apl, p_ref[...], preferred_element_type=f32)
                     + b1a_ref[...])
    h1 = jax.nn.relu(jnp.dot(t1, w1b_ref[...], preferred_element_type=f32)
                     + b1b_ref[...])

    z2 = jnp.dot(apl, h1, preferred_element_type=f32)
    t2 = jax.nn.relu(jnp.dot(z2, w2a_ref[...], preferred_element_type=f32)
                     + b2a_ref[...])
    h2 = jnp.dot(t2, w2b_ref[...], preferred_element_type=f32) + b2b_ref[...]

    s = jnp.sum(h2 * wp_ref[...], axis=1, keepdims=True)
    total = jnp.sum(s, axis=0, keepdims=True) + bp_ref[...]
    out_ref[...] = jax.nn.sigmoid(total)


@jax.jit
def _run(x, edge_index, pred_connectivity, W1a, b1a, W1b, b1b,
         W2a, b2a, W2b, b2b, Wp, bp):
    src = edge_index[0]
    dst = edge_index[1]
    pad = EP - E
    ds_p = jnp.concatenate([jnp.pad(dst, (0, pad)), jnp.pad(src, (0, pad))])
    pc_p = jnp.pad(pred_connectivity, (0, pad)).reshape(EP // 128, 128)

    a_flat = _build_a(ds_p, pc_p)
    p = pl.pallas_call(
        _proj_kernel,
        out_shape=jax.ShapeDtypeStruct((N, HID), jnp.float32),
    )(x, W1a)
    a2d = a_flat[:N * N].reshape(N, N)

    out = pl.pallas_call(
        _dense_kernel,
        out_shape=jax.ShapeDtypeStruct((1, 1), jnp.float32),
    )(a2d, p,
      b1a.reshape(1, HID), W1b, b1b.reshape(1, HID),
      W2a, b2a.reshape(1, OUT), W2b, b2b.reshape(1, OUT),
      Wp.reshape(N, OUT), bp.reshape(1, 1))
    return out.reshape(1)


def kernel(x, edge_index, pred_connectivity, W1a, b1a, W1b, b1b,
           W2a, b2a, W2b, b2b, Wp, bp):
    return _run(x, edge_index, pred_connectivity, W1a, b1a, W1b, b1b,
                W2a, b2a, W2b, b2b, Wp, bp)


# speedup vs baseline: 1.1587x; 1.0555x over previous
import functools

import jax
import jax.numpy as jnp
from jax import lax
from jax.experimental import pallas as pl
from jax.experimental.pallas import tpu as pltpu
from jax.experimental.pallas import tpu_sc as plsc

N = 19
E = 342
EP = 384
D_IN = 1025
HID = 256
OUT = 512
ROW = 128
AP = 2560
DUMP = N * ROW


def _build_a_body(ei_h, pc_h, zero_h, out_h,
                  ds_v, idx_v, pc_v, acc_sh, sem):
    del sem
    cid = lax.axis_index("c")
    sid = lax.axis_index("s")

    @pl.when((cid == 0) & (sid == 0))
    def _():
        pltpu.sync_copy(ei_h, ds_v)
        pltpu.sync_copy(pc_h, pc_v.at[pl.ds(0, E)])
        pltpu.sync_copy(zero_h, acc_sh)
        dump = jnp.full((16,), DUMP, jnp.int32)
        idx_v[2, pl.ds(86, 16)] = dump
        idx_v[2, pl.ds(102, 16)] = dump
        idx_v[2, pl.ds(112, 16)] = dump
        for c in range(22):
            off = c * 16 if c < 21 else E - 16
            d = ds_v[1, pl.ds(off, 16)]
            s = ds_v[0, pl.ds(off, 16)]
            idx_v[off // 128, pl.ds(off % 128, 16)] = d * ROW + s
        for j in range(EP // 128):
            pltpu.sync_copy(pc_v.at[pl.ds(j * 128, 128)],
                            acc_sh.at[idx_v.at[j]], add=True)
        pltpu.sync_copy(acc_sh, out_h)


_build_a = functools.partial(
    pl.kernel,
    out_type=jax.ShapeDtypeStruct((AP,), jnp.float32),
    mesh=plsc.VectorSubcoreMesh(core_axis_name="c", subcore_axis_name="s",
                                num_cores=1),
    scratch_types=[
        pltpu.VMEM((2, E), jnp.int32),
        pltpu.VMEM((EP // 128, 128), jnp.int32),
        pltpu.VMEM((EP,), jnp.float32),
        pltpu.VMEM_SHARED((AP,), jnp.float32),
        pltpu.SemaphoreType.DMA,
    ],
)(_build_a_body)


def _proj_kernel(x_ref, w1a_ref, out_ref):
    out_ref[...] = jnp.dot(x_ref[...], w1a_ref[...],
                           preferred_element_type=jnp.float32)


def _dense_kernel(a_ref, p_ref,
                  b1a_ref, w1b_ref, b1b_ref,
                  w2a_ref, b2a_ref, w2b_ref, b2b_ref,
                  wp_ref, bp_ref, out_ref):
    f32 = jnp.float32
    a = a_ref[...].reshape(AP // ROW, ROW)[:N, :N]
    eye = (jax.lax.broadcasted_iota(jnp.int32, (N, N), 0)
           == jax.lax.broadcasted_iota(jnp.int32, (N, N), 1)).astype(f32)
    apl = a + eye

    t1 = jax.nn.relu(jnp.dot(apl, p_ref[...], preferred_element_type=f32)
                     + b1a_ref[...])
    h1 = jax.nn.relu(jnp.dot(t1, w1b_ref[...], preferred_element_type=f32)
                     + b1b_ref[...])

    z2 = jnp.dot(apl, h1, preferred_element_type=f32)
    t2 = jax.nn.relu(jnp.dot(z2, w2a_ref[...], preferred_element_type=f32)
                     + b2a_ref[...])
    h2 = jnp.dot(t2, w2b_ref[...], preferred_element_type=f32) + b2b_ref[...]

    s = jnp.sum(h2 * wp_ref[...], axis=1, keepdims=True)
    total = jnp.sum(s, axis=0, keepdims=True) + bp_ref[...]
    out_ref[...] = jax.nn.sigmoid(total)


@jax.jit
def _run(x, edge_index, pred_connectivity, W1a, b1a, W1b, b1b,
         W2a, b2a, W2b, b2b, Wp, bp):
    a_flat = _build_a(edge_index, pred_connectivity,
                      jnp.zeros((AP,), jnp.float32))
    p = pl.pallas_call(
        _proj_kernel,
        out_shape=jax.ShapeDtypeStruct((N, HID), jnp.float32),
    )(x, W1a)

    out = pl.pallas_call(
        _dense_kernel,
        out_shape=jax.ShapeDtypeStruct((1, 1), jnp.float32),
    )(a_flat, p,
      b1a.reshape(1, HID), W1b, b1b.reshape(1, HID),
      W2a, b2a.reshape(1, OUT), W2b, b2b.reshape(1, OUT),
      Wp.reshape(N, OUT), bp.reshape(1, 1))
    return out.reshape(1)


def kernel(x, edge_index, pred_connectivity, W1a, b1a, W1b, b1b,
           W2a, b2a, W2b, b2b, Wp, bp):
    return _run(x, edge_index, pred_connectivity, W1a, b1a, W1b, b1b,
                W2a, b2a, W2b, b2b, Wp, bp)
